# Initial kernel scaffold; baseline (speedup 1.0000x reference)
import jax
import jax.numpy as jnp
from jax.experimental import pallas as pl


def kernel(source1, source2, dummy_vector, word1, word2):
    raise NotImplementedError("write your pallas kernel here")


# speedup vs baseline: 2.2006x; 2.2006x over previous
import functools

import jax
import jax.numpy as jnp
from jax import lax
from jax.experimental import pallas as pl
from jax.experimental.pallas import tpu as pltpu
from jax.experimental.pallas import tpu_sc as plsc

VOCAB = 100000
D = 64
B = 16384

NC = 2
NS = 16
NW = NC * NS
BPW = B // NW

_EPS = 1e-8


def _sc_gather_pair():
    mesh = plsc.VectorSubcoreMesh(core_axis_name="c", subcore_axis_name="s")

    @functools.partial(
        pl.kernel,
        out_type=(
            jax.ShapeDtypeStruct((B, D), jnp.float32),
            jax.ShapeDtypeStruct((B, D), jnp.float32),
        ),
        mesh=mesh,
        scratch_types=[
            pltpu.VMEM((BPW,), jnp.int32),
            pltpu.VMEM((BPW, D), jnp.float32),
            pltpu.VMEM((BPW,), jnp.int32),
            pltpu.VMEM((BPW, D), jnp.float32),
            pltpu.SemaphoreType.DMA,
            pltpu.SemaphoreType.DMA,
        ],
        compiler_params=pltpu.CompilerParams(use_tc_tiling_on_sc=False),
    )
    def gather2(t1, idx1_hbm, t2, idx2_hbm, out1, out2,
                idx1_v, rows1_v, idx2_v, rows2_v, sem1, sem2):
        wid = lax.axis_index("s") * NC + lax.axis_index("c")
        base = wid * BPW
        pltpu.sync_copy(idx1_hbm.at[pl.ds(base, BPW)], idx1_v)
        pltpu.sync_copy(idx2_hbm.at[pl.ds(base, BPW)], idx2_v)
        c1 = pltpu.async_copy(t1.at[idx1_v], rows1_v, sem1)
        c2 = pltpu.async_copy(t2.at[idx2_v], rows2_v, sem2)
        c1.wait()
        c2.wait()
        pltpu.sync_copy(rows1_v, out1.at[pl.ds(base, BPW)])
        pltpu.sync_copy(rows2_v, out2.at[pl.ds(base, BPW)])

    return gather2


def _tc_body(e1_ref, e2_ref, m_ref, o_ref):
    m = m_ref[...]
    w1 = jnp.dot(e1_ref[...], m, preferred_element_type=jnp.float32)
    w2 = jnp.dot(e2_ref[...], m, preferred_element_type=jnp.float32)
    num = jnp.sum(w1 * w2, axis=1)
    n1 = jnp.sum(w1 * w1, axis=1)
    n2 = jnp.sum(w2 * w2, axis=1)
    o_ref[...] = num / (jnp.sqrt(n1) * jnp.sqrt(n2) + _EPS)


def _tc_bind_cosine(e1, e2, m, block_b=2048):
    grid = B // block_b
    return pl.pallas_call(
        _tc_body,
        grid=(grid,),
        in_specs=[
            pl.BlockSpec((block_b, D), lambda i: (i, 0)),
            pl.BlockSpec((block_b, D), lambda i: (i, 0)),
            pl.BlockSpec((D, D), lambda i: (0, 0)),
        ],
        out_specs=pl.BlockSpec((block_b,), lambda i: (i,)),
        out_shape=jax.ShapeDtypeStruct((B,), jnp.float32),
    )(e1, e2, m)


def _circulant(d):
    j = jnp.arange(D)[:, None]
    k = jnp.arange(D)[None, :]
    return jnp.take(d, (k - j) % D, axis=0)


def kernel(source1, source2, dummy_vector, word1, word2):
    idx1 = word1.astype(jnp.int32)
    idx2 = word2.astype(jnp.int32)
    e1, e2 = _sc_gather_pair()(source1, idx1, source2, idx2)
    m = _circulant(dummy_vector)
    return _tc_bind_cosine(e1, e2, m)
